# Initial kernel scaffold; baseline (speedup 1.0000x reference)
import jax
import jax.numpy as jnp
from jax.experimental import pallas as pl


def kernel(x, edge_index, W1, a_src1, a_dst1, b1, Wskip1, bskip1, W2, a_src2, a_dst2, b2, Wskip2, bskip2, W3, a_src3, a_dst3, b3):
    raise NotImplementedError("write your pallas kernel here")


# speedup vs baseline: 1.0297x; 1.0297x over previous
import jax
import jax.numpy as jnp
from jax.experimental import pallas as pl

NUM_HEADS = 8
IN_CH = 128
H1 = 64
D1 = H1 * NUM_HEADS
H2 = 128
HEADS2 = NUM_HEADS // 2
D2 = H2 * HEADS2
NUM_CLASSES = 64
D3_IN = D2 + D1
N_NODES = 10000
N_EDGES = 320000


def _mm_kernel(x_ref, w_ref, o_ref):
    o_ref[...] = jnp.dot(x_ref[...], w_ref[...], preferred_element_type=jnp.float32)


def _pallas_mm(x, w):
    n, k = x.shape
    k2, m = w.shape
    blk = 1000
    return pl.pallas_call(
        _mm_kernel,
        grid=(n // blk,),
        in_specs=[
            pl.BlockSpec((blk, k), lambda i: (i, 0)),
            pl.BlockSpec((k, m), lambda i: (0, 0)),
        ],
        out_specs=pl.BlockSpec((blk, m), lambda i: (i, 0)),
        out_shape=jax.ShapeDtypeStruct((n, m), jnp.float32),
    )(x, w)


def _gat(x, edge_index, W, a_src, a_dst, b, heads, out_ch):
    N = x.shape[0]
    h = _pallas_mm(x, W).reshape(N, heads, out_ch)
    src = edge_index[0]
    dst = edge_index[1]
    alpha_src = jnp.sum(h * a_src[None, :, :], axis=-1)
    alpha_dst = jnp.sum(h * a_dst[None, :, :], axis=-1)
    e = jax.nn.leaky_relu(alpha_src[src] + alpha_dst[dst], negative_slope=0.2)
    emax = jax.ops.segment_max(e, dst, num_segments=N)
    emax = jnp.where(jnp.isfinite(emax), emax, 0.0)
    ex = jnp.exp(e - emax[dst])
    den = jax.ops.segment_sum(ex, dst, num_segments=N)
    alpha = ex / (den[dst] + 1e-16)
    msg = h[src] * alpha[:, :, None]
    out = jax.ops.segment_sum(msg, dst, num_segments=N)
    return out.reshape(N, heads * out_ch) + b


def kernel(x, edge_index, W1, a_src1, a_dst1, b1, Wskip1, bskip1,
           W2, a_src2, a_dst2, b2, Wskip2, bskip2,
           W3, a_src3, a_dst3, b3):
    x1 = _gat(x, edge_index, W1, a_src1, a_dst1, b1, NUM_HEADS, H1)
    x_skip1 = _pallas_mm(x, Wskip1) + bskip1
    x_skip2 = _pallas_mm(x1, Wskip2) + bskip2
    x3 = jnp.concatenate([x_skip1, x_skip2], axis=1)
    return _gat(x3, edge_index, W3, a_src3, a_dst3, b3, 1, NUM_CLASSES)


# speedup vs baseline: 11.5823x; 11.2487x over previous
import functools

import jax
import jax.numpy as jnp
from jax import lax
from jax.experimental import pallas as pl
from jax.experimental.pallas import tpu as pltpu
from jax.experimental.pallas import tpu_sc as plsc

NUM_HEADS = 8
IN_CH = 128
H1 = 64
D1 = H1 * NUM_HEADS
NUM_CLASSES = 64
D3_IN = 1024
N = 10000
E = 320000

NC = 2
NS = 16
NW = NC * NS
L = 16

EPW = E // NW
BA = 80
NB = EPW // BA
NP = 10240
RPT = NP // NS
ZR = 128

BLK = 1000


def _tc1_body(x_ref, w1_ref, as_ref, ad_ref, wsk_ref, h1_ref, asn_ref,
              adn_ref, xs1_ref):
    xb = x_ref[...]
    h = jnp.dot(xb, w1_ref[...], preferred_element_type=jnp.float32)
    h1_ref[...] = h
    asn_ref[...] = jnp.dot(h, as_ref[...], preferred_element_type=jnp.float32)
    adn_ref[...] = jnp.dot(h, ad_ref[...], preferred_element_type=jnp.float32)
    xs1_ref[...] = jnp.dot(xb, wsk_ref[...], preferred_element_type=jnp.float32)


def _tc1(x, W1, A1s, A1d, Wskip1):
    return pl.pallas_call(
        _tc1_body,
        grid=(N // BLK,),
        in_specs=[
            pl.BlockSpec((BLK, IN_CH), lambda i: (i, 0)),
            pl.BlockSpec((IN_CH, D1), lambda i: (0, 0)),
            pl.BlockSpec((D1, L), lambda i: (0, 0)),
            pl.BlockSpec((D1, L), lambda i: (0, 0)),
            pl.BlockSpec((IN_CH, D1), lambda i: (0, 0)),
        ],
        out_specs=[
            pl.BlockSpec((BLK, D1), lambda i: (i, 0)),
            pl.BlockSpec((BLK, L), lambda i: (i, 0)),
            pl.BlockSpec((BLK, L), lambda i: (i, 0)),
            pl.BlockSpec((BLK, D1), lambda i: (i, 0)),
        ],
        out_shape=[
            jax.ShapeDtypeStruct((N, D1), jnp.float32),
            jax.ShapeDtypeStruct((N, L), jnp.float32),
            jax.ShapeDtypeStruct((N, L), jnp.float32),
            jax.ShapeDtypeStruct((N, D1), jnp.float32),
        ],
    )(x, W1, A1s, A1d, Wskip1)


def _tc2_body(n0_ref, n1_ref, n2_ref, n3_ref, den_ref, b1_ref, xs1_ref,
              wsk2_ref, bsk2_ref, w3_ref, a3s_ref, a3d_ref,
              h3_ref, asn3_ref, adn3_ref):
    den = den_ref[0] + den_ref[1] + 1e-16
    parts = []
    for c, nref in enumerate((n0_ref, n1_ref, n2_ref, n3_ref)):
        num = nref[0] + nref[1]
        d2 = den[:, 2 * c:2 * c + 2]
        drep = jnp.broadcast_to(d2[:, :, None], (BLK, 2, H1)).reshape(BLK, 128)
        parts.append(num / drep)
    x1 = jnp.concatenate(parts, axis=1) + b1_ref[...][None, :]
    xs2 = jnp.dot(x1, wsk2_ref[...], preferred_element_type=jnp.float32)
    xs2 = xs2 + bsk2_ref[...][None, :]
    x3 = jnp.concatenate([xs1_ref[...], xs2], axis=1)
    h3 = jnp.dot(x3, w3_ref[...], preferred_element_type=jnp.float32)
    h3_ref[...] = h3
    asn3_ref[...] = jnp.dot(h3, a3s_ref[...], preferred_element_type=jnp.float32)
    adn3_ref[...] = jnp.dot(h3, a3d_ref[...], preferred_element_type=jnp.float32)


def _tc2(nums, den, b1, xs1, Wskip2, bskip2, W3, A3s, A3d):
    num_specs = [pl.BlockSpec((2, BLK, 128), lambda i: (0, i, 0))
                 for _ in range(4)]
    return pl.pallas_call(
        _tc2_body,
        grid=(N // BLK,),
        in_specs=num_specs + [
            pl.BlockSpec((2, BLK, L), lambda i: (0, i, 0)),
            pl.BlockSpec((D1,), lambda i: (0,)),
            pl.BlockSpec((BLK, D1), lambda i: (i, 0)),
            pl.BlockSpec((D1, D1), lambda i: (0, 0)),
            pl.BlockSpec((D1,), lambda i: (0,)),
            pl.BlockSpec((D3_IN, NUM_CLASSES), lambda i: (0, 0)),
            pl.BlockSpec((NUM_CLASSES, L), lambda i: (0, 0)),
            pl.BlockSpec((NUM_CLASSES, L), lambda i: (0, 0)),
        ],
        out_specs=[
            pl.BlockSpec((BLK, NUM_CLASSES), lambda i: (i, 0)),
            pl.BlockSpec((BLK, L), lambda i: (i, 0)),
            pl.BlockSpec((BLK, L), lambda i: (i, 0)),
        ],
        out_shape=[
            jax.ShapeDtypeStruct((N, NUM_CLASSES), jnp.float32),
            jax.ShapeDtypeStruct((N, L), jnp.float32),
            jax.ShapeDtypeStruct((N, L), jnp.float32),
        ],
    )(*nums, den, b1, xs1, Wskip2, bskip2, W3, A3s, A3d)


def _tc3_body(num_ref, den_ref, b3_ref, out_ref):
    den = den_ref[0] + den_ref[1] + 1e-16
    d = den[:, 0:1]
    drep = jnp.broadcast_to(d, (BLK, NUM_CLASSES))
    num = num_ref[0] + num_ref[1]
    out_ref[...] = num / drep + b3_ref[...][None, :]


def _tc3(num3, den3, b3):
    return pl.pallas_call(
        _tc3_body,
        grid=(N // BLK,),
        in_specs=[
            pl.BlockSpec((2, BLK, NUM_CLASSES), lambda i: (0, i, 0)),
            pl.BlockSpec((2, BLK, L), lambda i: (0, i, 0)),
            pl.BlockSpec((NUM_CLASSES,), lambda i: (0,)),
        ],
        out_specs=pl.BlockSpec((BLK, NUM_CLASSES), lambda i: (i, 0)),
        out_shape=jax.ShapeDtypeStruct((N, NUM_CLASSES), jnp.float32),
    )(num3, den3, b3)


_MESH = plsc.VectorSubcoreMesh(core_axis_name="c", subcore_axis_name="s",
                               num_cores=NC, num_subcores=NS)


def _attn_body(asn_h, adn_h, src_h, dst_h, w_h, den_h,
               src_v, dst_v, as_v, ad_v, w_v, z_v, den_acc, sem):
    cid = lax.axis_index("c")
    sid = lax.axis_index("s")
    wid = sid * NC + cid
    base = wid * EPW

    def zrow(i, carry):
        z_v[i, :] = jnp.zeros((L,), jnp.float32)
        return carry
    lax.fori_loop(0, ZR, zrow, 0)

    def zcopy(i, carry):
        pltpu.sync_copy(z_v, den_acc.at[pl.ds(sid * RPT + i * ZR, ZR)])
        return carry
    lax.fori_loop(0, RPT // ZR, zcopy, 0)
    plsc.subcore_barrier()

    def batch(i, carry):
        off = base + i * BA
        pltpu.sync_copy(src_h.at[pl.ds(off, BA)], src_v)
        pltpu.sync_copy(dst_h.at[pl.ds(off, BA)], dst_v)
        pltpu.async_copy(asn_h.at[src_v], as_v, sem).wait()
        pltpu.async_copy(adn_h.at[dst_v], ad_v, sem).wait()

        def edge(j, c2):
            z = as_v[j, :] + ad_v[j, :]
            w_v[j, :] = jnp.exp(jnp.maximum(z, 0.2 * z))
            return c2
        lax.fori_loop(0, BA, edge, 0)
        pltpu.sync_copy(w_v, w_h.at[pl.ds(off, BA)])
        pltpu.sync_copy(w_v, den_acc.at[dst_v], add=True)
        return carry
    lax.fori_loop(0, NB, batch, 0)
    plsc.subcore_barrier()

    pltpu.sync_copy(den_acc.at[pl.ds(sid * RPT, RPT)],
                    den_h.at[pl.ds(cid * NP + sid * RPT, RPT)])


def _attn(asn, adn, src, dst):
    return pl.kernel(
        _attn_body,
        out_type=[
            jax.ShapeDtypeStruct((E, L), jnp.float32),
            jax.ShapeDtypeStruct((2 * NP, L), jnp.float32),
        ],
        mesh=_MESH,
        compiler_params=pltpu.CompilerParams(use_tc_tiling_on_sc=False,
                                             needs_layout_passes=False),
        scratch_types=[
            pltpu.VMEM((BA,), jnp.int32),
            pltpu.VMEM((BA,), jnp.int32),
            pltpu.VMEM((BA, L), jnp.float32),
            pltpu.VMEM((BA, L), jnp.float32),
            pltpu.VMEM((BA, L), jnp.float32),
            pltpu.VMEM((ZR, L), jnp.float32),
            pltpu.VMEM_SHARED((NP, L), jnp.float32),
            pltpu.SemaphoreType.DMA,
        ],
    )(asn, adn, src, dst)


def _make_agg_body(F, head0, head1):
    nreg = F // L

    def agg_body(hc_h, src_h, dst_h, w_h, num_h,
                 src_v, dst_v, rows_v, msg_v, w_v, z_v, num_acc, sem):
        cid = lax.axis_index("c")
        sid = lax.axis_index("s")
        wid = sid * NC + cid
        base = wid * EPW
        h0v = jnp.full((L,), head0, jnp.int32)
        h1v = jnp.full((L,), head1, jnp.int32)

        def zrow(i, carry):
            for k in range(nreg):
                z_v[i, pl.ds(L * k, L)] = jnp.zeros((L,), jnp.float32)
            return carry
        lax.fori_loop(0, ZR, zrow, 0)

        def zcopy(i, carry):
            pltpu.sync_copy(z_v, num_acc.at[pl.ds(sid * RPT + i * ZR, ZR)])
            return carry
        lax.fori_loop(0, RPT // ZR, zcopy, 0)
        plsc.subcore_barrier()

        def batch(i, carry):
            off = base + i * BA
            pltpu.sync_copy(src_h.at[pl.ds(off, BA)], src_v)
            pltpu.sync_copy(dst_h.at[pl.ds(off, BA)], dst_v)
            pltpu.async_copy(hc_h.at[src_v], rows_v, sem).wait()
            pltpu.sync_copy(w_h.at[pl.ds(off, BA)], w_v)

            def edge(j, c2):
                jj = jnp.full((L,), j, jnp.int32)
                w0 = plsc.load_gather(w_v, [jj, h0v])
                w1 = plsc.load_gather(w_v, [jj, h1v])
                for k in range(nreg):
                    wk = w0 if k < nreg // 2 else w1
                    msg_v[j, pl.ds(L * k, L)] = rows_v[j, pl.ds(L * k, L)] * wk
                return c2
            lax.fori_loop(0, BA, edge, 0)
            pltpu.sync_copy(msg_v, num_acc.at[dst_v], add=True)
            return carry
        lax.fori_loop(0, NB, batch, 0)
        plsc.subcore_barrier()

        pltpu.sync_copy(num_acc.at[pl.ds(sid * RPT, RPT)],
                        num_h.at[pl.ds(cid * NP + sid * RPT, RPT)])

    return agg_body


def _agg(hc, src, dst, w, F, head0, head1):
    return pl.kernel(
        _make_agg_body(F, head0, head1),
        out_type=jax.ShapeDtypeStruct((2 * NP, F), jnp.float32),
        mesh=_MESH,
        compiler_params=pltpu.CompilerParams(use_tc_tiling_on_sc=False,
                                             needs_layout_passes=False),
        scratch_types=[
            pltpu.VMEM((BA,), jnp.int32),
            pltpu.VMEM((BA,), jnp.int32),
            pltpu.VMEM((BA, F), jnp.float32),
            pltpu.VMEM((BA, F), jnp.float32),
            pltpu.VMEM((BA, L), jnp.float32),
            pltpu.VMEM((ZR, F), jnp.float32),
            pltpu.VMEM_SHARED((NP, F), jnp.float32),
            pltpu.SemaphoreType.DMA,
        ],
    )(hc, src, dst, w)


def _blockdiag(a, heads, ch):
    eye = jnp.eye(heads, dtype=jnp.float32)
    A = (eye[:, None, :] * a[:, :, None]).reshape(heads * ch, heads)
    return jnp.concatenate(
        [A, jnp.zeros((heads * ch, L - heads), jnp.float32)], axis=1)


def kernel(x, edge_index, W1, a_src1, a_dst1, b1, Wskip1, bskip1,
           W2, a_src2, a_dst2, b2, Wskip2, bskip2,
           W3, a_src3, a_dst3, b3):
    src = edge_index[0].astype(jnp.int32)
    dst = edge_index[1].astype(jnp.int32)

    A1s = _blockdiag(a_src1, NUM_HEADS, H1)
    A1d = _blockdiag(a_dst1, NUM_HEADS, H1)
    A3s = _blockdiag(a_src3, 1, NUM_CLASSES)
    A3d = _blockdiag(a_dst3, 1, NUM_CLASSES)

    h1, asn1, adn1, xs1 = _tc1(x, W1, A1s, A1d, Wskip1)
    xs1 = xs1 + bskip1[None, :]

    w1, den1 = _attn(asn1, adn1, src, dst)

    nums = []
    for c in range(4):
        hc = lax.slice(h1, (0, 128 * c), (N, 128 * (c + 1)))
        num = _agg(hc, src, dst, w1, 128, 2 * c, 2 * c + 1)
        nums.append(num.reshape(2, NP, 128)[:, :N, :])

    den1r = den1.reshape(2, NP, L)[:, :N, :]
    h3, asn3, adn3 = _tc2(nums, den1r, b1, xs1, Wskip2, bskip2, W3, A3s, A3d)

    w3, den3 = _attn(asn3, adn3, src, dst)
    num3 = _agg(h3, src, dst, w3, NUM_CLASSES, 0, 0)

    out = _tc3(num3.reshape(2, NP, NUM_CLASSES)[:, :N, :],
               den3.reshape(2, NP, L)[:, :N, :], b3)
    return out


# speedup vs baseline: 22.6463x; 1.9553x over previous
import jax
import jax.numpy as jnp
from jax import lax
from jax.experimental import pallas as pl
from jax.experimental.pallas import tpu as pltpu
from jax.experimental.pallas import tpu_sc as plsc

NUM_HEADS = 8
IN_CH = 128
H1 = 64
D1 = H1 * NUM_HEADS
NUM_CLASSES = 64
D3_IN = 1024
N = 10000
E = 320000

NC = 2
NS = 16
NW = NC * NS
L = 16

EPW = E // NW
BA = 80
NB = EPW // BA
NP = 10240
RPT = NP // NS

BLK = 1000


def _tc1_body(x_ref, w1_ref, as_ref, ad_ref, wsk_ref, h1_ref, asn_ref,
              adn_ref, xs1_ref):
    xb = x_ref[...]
    h = jnp.dot(xb, w1_ref[...], preferred_element_type=jnp.float32)
    h1_ref[...] = h
    asn_ref[...] = jnp.dot(h, as_ref[...], preferred_element_type=jnp.float32)
    adn_ref[...] = jnp.dot(h, ad_ref[...], preferred_element_type=jnp.float32)
    xs1_ref[...] = jnp.dot(xb, wsk_ref[...], preferred_element_type=jnp.float32)


def _tc1(x, W1, A1s, A1d, Wskip1):
    return pl.pallas_call(
        _tc1_body,
        grid=(N // BLK,),
        in_specs=[
            pl.BlockSpec((BLK, IN_CH), lambda i: (i, 0)),
            pl.BlockSpec((IN_CH, D1), lambda i: (0, 0)),
            pl.BlockSpec((D1, L), lambda i: (0, 0)),
            pl.BlockSpec((D1, L), lambda i: (0, 0)),
            pl.BlockSpec((IN_CH, D1), lambda i: (0, 0)),
        ],
        out_specs=[
            pl.BlockSpec((BLK, D1), lambda i: (i, 0)),
            pl.BlockSpec((BLK, L), lambda i: (i, 0)),
            pl.BlockSpec((BLK, L), lambda i: (i, 0)),
            pl.BlockSpec((BLK, D1), lambda i: (i, 0)),
        ],
        out_shape=[
            jax.ShapeDtypeStruct((N, D1), jnp.float32),
            jax.ShapeDtypeStruct((N, L), jnp.float32),
            jax.ShapeDtypeStruct((N, L), jnp.float32),
            jax.ShapeDtypeStruct((N, D1), jnp.float32),
        ],
    )(x, W1, A1s, A1d, Wskip1)


def _tc2_body(n0_ref, n1_ref, n2_ref, n3_ref, den_ref, b1_ref, xs1_ref,
              wsk2_ref, bsk2_ref, w3_ref, a3s_ref, a3d_ref,
              h3_ref, asn3_ref, adn3_ref):
    den = den_ref[0] + den_ref[1] + 1e-16
    parts = []
    for c, nref in enumerate((n0_ref, n1_ref, n2_ref, n3_ref)):
        num = nref[0] + nref[1]
        d2 = den[:, 2 * c:2 * c + 2]
        drep = jnp.broadcast_to(d2[:, :, None], (BLK, 2, H1)).reshape(BLK, 128)
        parts.append(num / drep)
    x1 = jnp.concatenate(parts, axis=1) + b1_ref[...][None, :]
    xs2 = jnp.dot(x1, wsk2_ref[...], preferred_element_type=jnp.float32)
    xs2 = xs2 + bsk2_ref[...][None, :]
    x3 = jnp.concatenate([xs1_ref[...], xs2], axis=1)
    h3 = jnp.dot(x3, w3_ref[...], preferred_element_type=jnp.float32)
    h3_ref[...] = h3
    asn3_ref[...] = jnp.dot(h3, a3s_ref[...], preferred_element_type=jnp.float32)
    adn3_ref[...] = jnp.dot(h3, a3d_ref[...], preferred_element_type=jnp.float32)


def _tc2(nums, den, b1, xs1, Wskip2, bskip2, W3, A3s, A3d):
    num_specs = [pl.BlockSpec((2, BLK, 128), lambda i: (0, i, 0))
                 for _ in range(4)]
    return pl.pallas_call(
        _tc2_body,
        grid=(N // BLK,),
        in_specs=num_specs + [
            pl.BlockSpec((2, BLK, L), lambda i: (0, i, 0)),
            pl.BlockSpec((D1,), lambda i: (0,)),
            pl.BlockSpec((BLK, D1), lambda i: (i, 0)),
            pl.BlockSpec((D1, D1), lambda i: (0, 0)),
            pl.BlockSpec((D1,), lambda i: (0,)),
            pl.BlockSpec((D3_IN, NUM_CLASSES), lambda i: (0, 0)),
            pl.BlockSpec((NUM_CLASSES, L), lambda i: (0, 0)),
            pl.BlockSpec((NUM_CLASSES, L), lambda i: (0, 0)),
        ],
        out_specs=[
            pl.BlockSpec((BLK, NUM_CLASSES), lambda i: (i, 0)),
            pl.BlockSpec((BLK, L), lambda i: (i, 0)),
            pl.BlockSpec((BLK, L), lambda i: (i, 0)),
        ],
        out_shape=[
            jax.ShapeDtypeStruct((N, NUM_CLASSES), jnp.float32),
            jax.ShapeDtypeStruct((N, L), jnp.float32),
            jax.ShapeDtypeStruct((N, L), jnp.float32),
        ],
    )(*nums, den, b1, xs1, Wskip2, bskip2, W3, A3s, A3d)


def _tc3_body(num_ref, den_ref, b3_ref, out_ref):
    den = den_ref[0] + den_ref[1] + 1e-16
    d = den[:, 0:1]
    drep = jnp.broadcast_to(d, (BLK, NUM_CLASSES))
    num = num_ref[0] + num_ref[1]
    out_ref[...] = num / drep + b3_ref[...][None, :]


def _tc3(num3, den3, b3):
    return pl.pallas_call(
        _tc3_body,
        grid=(N // BLK,),
        in_specs=[
            pl.BlockSpec((2, BLK, NUM_CLASSES), lambda i: (0, i, 0)),
            pl.BlockSpec((2, BLK, L), lambda i: (0, i, 0)),
            pl.BlockSpec((NUM_CLASSES,), lambda i: (0,)),
        ],
        out_specs=pl.BlockSpec((BLK, NUM_CLASSES), lambda i: (i, 0)),
        out_shape=jax.ShapeDtypeStruct((N, NUM_CLASSES), jnp.float32),
    )(num3, den3, b3)


_MESH = plsc.VectorSubcoreMesh(core_axis_name="c", subcore_axis_name="s",
                               num_cores=NC, num_subcores=NS)
_SC_PARAMS = pltpu.CompilerParams(use_tc_tiling_on_sc=False,
                                  needs_layout_passes=False)


def _zero_acc(zsrc, acc, sid, nreg):
    def zrow(i, carry):
        for k in range(nreg):
            zsrc[i, pl.ds(L * k, L)] = jnp.zeros((L,), jnp.float32)
        return carry
    lax.fori_loop(0, BA, zrow, 0)

    def zcopy(i, carry):
        pltpu.sync_copy(zsrc, acc.at[pl.ds(sid * RPT + i * BA, BA)])
        return carry
    lax.fori_loop(0, RPT // BA, zcopy, 0)


def _attn_body(asn_h, adn_h, src_h, dst_h, w_h, den_h,
               src_v, dst_v, as_v, ad_v, w_v, den_acc,
               isem, gsem, wsem, ssem):
    cid = lax.axis_index("c")
    sid = lax.axis_index("s")
    wid = sid * NC + cid
    base = wid * EPW

    def load_idx(i):
        s = lax.rem(i, 4)
        pltpu.async_copy(src_h.at[pl.ds(base + i * BA, BA)], src_v.at[s],
                         isem.at[s])
        pltpu.async_copy(dst_h.at[pl.ds(base + i * BA, BA)], dst_v.at[s],
                         isem.at[s])

    def wait_idx(i):
        s = lax.rem(i, 4)
        for _ in range(2):
            pltpu.make_async_copy(src_h.at[pl.ds(0, BA)], src_v.at[s],
                                  isem.at[s]).wait()

    def load_rows(i):
        s = lax.rem(i, 4)
        b = lax.rem(i, 2)
        pltpu.async_copy(asn_h.at[src_v.at[s]], as_v.at[b], gsem.at[b])
        pltpu.async_copy(adn_h.at[dst_v.at[s]], ad_v.at[b], gsem.at[b])

    _zero_acc(w_v.at[0], den_acc, sid, 1)
    load_idx(0)
    load_idx(1)
    wait_idx(0)
    load_rows(0)
    plsc.subcore_barrier()

    def batch(i, carry):
        s = lax.rem(i, 4)
        b = lax.rem(i, 2)

        @pl.when(i + 1 < NB)
        def _():
            wait_idx(i + 1)
            load_rows(i + 1)

        @pl.when(i >= 2)
        def _():
            pltpu.make_async_copy(w_v.at[b], w_h.at[pl.ds(0, BA)],
                                  wsem.at[b]).wait()
            pltpu.make_async_copy(w_v.at[b], den_acc.at[pl.ds(0, BA)],
                                  ssem.at[b]).wait()

        @pl.when(i + 2 < NB)
        def _():
            load_idx(i + 2)

        for _ in range(2):
            pltpu.make_async_copy(asn_h.at[pl.ds(0, BA)], as_v.at[b],
                                  gsem.at[b]).wait()

        def edge(j, c2):
            z = as_v[b, j, :] + ad_v[b, j, :]
            w_v[b, j, :] = jnp.exp(jnp.maximum(z, 0.2 * z))
            return c2
        lax.fori_loop(0, BA, edge, 0)
        pltpu.async_copy(w_v.at[b], w_h.at[pl.ds(base + i * BA, BA)],
                         wsem.at[b])
        pltpu.async_copy(w_v.at[b], den_acc.at[dst_v.at[s]], ssem.at[b],
                         add=True)
        return carry
    lax.fori_loop(0, NB, batch, 0)

    for b in range(2):
        pltpu.make_async_copy(w_v.at[b], w_h.at[pl.ds(0, BA)],
                              wsem.at[b]).wait()
        pltpu.make_async_copy(w_v.at[b], den_acc.at[pl.ds(0, BA)],
                              ssem.at[b]).wait()
    plsc.subcore_barrier()

    pltpu.sync_copy(den_acc.at[pl.ds(sid * RPT, RPT)],
                    den_h.at[pl.ds(cid * NP + sid * RPT, RPT)])


def _attn(asn, adn, src, dst):
    return pl.kernel(
        _attn_body,
        out_type=[
            jax.ShapeDtypeStruct((E, L), jnp.float32),
            jax.ShapeDtypeStruct((2 * NP, L), jnp.float32),
        ],
        mesh=_MESH,
        compiler_params=_SC_PARAMS,
        scratch_types=[
            pltpu.VMEM((4, BA), jnp.int32),
            pltpu.VMEM((4, BA), jnp.int32),
            pltpu.VMEM((2, BA, L), jnp.float32),
            pltpu.VMEM((2, BA, L), jnp.float32),
            pltpu.VMEM((2, BA, L), jnp.float32),
            pltpu.VMEM_SHARED((NP, L), jnp.float32),
            pltpu.SemaphoreType.DMA((4,)),
            pltpu.SemaphoreType.DMA((2,)),
            pltpu.SemaphoreType.DMA((2,)),
            pltpu.SemaphoreType.DMA((2,)),
        ],
    )(asn, adn, src, dst)


def _make_agg_body(F, head0, head1):
    nreg = F // L

    def agg_body(hc_h, src_h, dst_h, w_h, num_h,
                 src_v, dst_v, rows_v, msg_v, w_v, num_acc,
                 isem, gsem, lsem, ssem):
        cid = lax.axis_index("c")
        sid = lax.axis_index("s")
        wid = sid * NC + cid
        base = wid * EPW
        h0v = jnp.full((L,), head0, jnp.int32)
        h1v = jnp.full((L,), head1, jnp.int32)

        def load_idx(i):
            s = lax.rem(i, 4)
            pltpu.async_copy(src_h.at[pl.ds(base + i * BA, BA)], src_v.at[s],
                             isem.at[s])
            pltpu.async_copy(dst_h.at[pl.ds(base + i * BA, BA)], dst_v.at[s],
                             isem.at[s])

        def wait_idx(i):
            s = lax.rem(i, 4)
            for _ in range(2):
                pltpu.make_async_copy(src_h.at[pl.ds(0, BA)], src_v.at[s],
                                      isem.at[s]).wait()

        def load_rows(i):
            s = lax.rem(i, 4)
            b = lax.rem(i, 2)
            pltpu.async_copy(hc_h.at[src_v.at[s]], rows_v.at[b], gsem.at[b])
            pltpu.async_copy(w_h.at[pl.ds((base + i * BA), BA)], w_v.at[b],
                             lsem.at[b])

        _zero_acc(msg_v.at[0], num_acc, sid, nreg)
        load_idx(0)
        load_idx(1)
        wait_idx(0)
        load_rows(0)
        plsc.subcore_barrier()

        def batch(i, carry):
            s = lax.rem(i, 4)
            b = lax.rem(i, 2)

            @pl.when(i + 1 < NB)
            def _():
                wait_idx(i + 1)
                load_rows(i + 1)

            @pl.when(i >= 2)
            def _():
                pltpu.make_async_copy(msg_v.at[b], num_acc.at[pl.ds(0, BA)],
                                      ssem.at[b]).wait()

            @pl.when(i + 2 < NB)
            def _():
                load_idx(i + 2)

            pltpu.make_async_copy(hc_h.at[pl.ds(0, BA)], rows_v.at[b],
                                  gsem.at[b]).wait()
            pltpu.make_async_copy(w_h.at[pl.ds(0, BA)], w_v.at[b],
                                  lsem.at[b]).wait()

            def edge(j, c2):
                jj = jnp.full((L,), j, jnp.int32)
                w0 = plsc.load_gather(w_v.at[b], [jj, h0v])
                w1 = plsc.load_gather(w_v.at[b], [jj, h1v])
                for k in range(nreg):
                    wk = w0 if k < nreg // 2 else w1
                    msg_v[b, j, pl.ds(L * k, L)] = \
                        rows_v[b, j, pl.ds(L * k, L)] * wk
                return c2
            lax.fori_loop(0, BA, edge, 0)
            pltpu.async_copy(msg_v.at[b], num_acc.at[dst_v.at[s]],
                             ssem.at[b], add=True)
            return carry
        lax.fori_loop(0, NB, batch, 0)

        for b in range(2):
            pltpu.make_async_copy(msg_v.at[b], num_acc.at[pl.ds(0, BA)],
                                  ssem.at[b]).wait()
        plsc.subcore_barrier()

        pltpu.sync_copy(num_acc.at[pl.ds(sid * RPT, RPT)],
                        num_h.at[pl.ds(cid * NP + sid * RPT, RPT)])

    return agg_body


def _agg(hc, src, dst, w, F, head0, head1):
    return pl.kernel(
        _make_agg_body(F, head0, head1),
        out_type=jax.ShapeDtypeStruct((2 * NP, F), jnp.float32),
        mesh=_MESH,
        compiler_params=_SC_PARAMS,
        scratch_types=[
            pltpu.VMEM((4, BA), jnp.int32),
            pltpu.VMEM((4, BA), jnp.int32),
            pltpu.VMEM((2, BA, F), jnp.float32),
            pltpu.VMEM((2, BA, F), jnp.float32),
            pltpu.VMEM((2, BA, L), jnp.float32),
            pltpu.VMEM_SHARED((NP, F), jnp.float32),
            pltpu.SemaphoreType.DMA((4,)),
            pltpu.SemaphoreType.DMA((2,)),
            pltpu.SemaphoreType.DMA((2,)),
            pltpu.SemaphoreType.DMA((2,)),
        ],
    )(hc, src, dst, w)


def _blockdiag(a, heads, ch):
    eye = jnp.eye(heads, dtype=jnp.float32)
    A = (eye[:, None, :] * a[:, :, None]).reshape(heads * ch, heads)
    return jnp.concatenate(
        [A, jnp.zeros((heads * ch, L - heads), jnp.float32)], axis=1)


def kernel(x, edge_index, W1, a_src1, a_dst1, b1, Wskip1, bskip1,
           W2, a_src2, a_dst2, b2, Wskip2, bskip2,
           W3, a_src3, a_dst3, b3):
    src = edge_index[0].astype(jnp.int32)
    dst = edge_index[1].astype(jnp.int32)

    A1s = _blockdiag(a_src1, NUM_HEADS, H1)
    A1d = _blockdiag(a_dst1, NUM_HEADS, H1)
    A3s = _blockdiag(a_src3, 1, NUM_CLASSES)
    A3d = _blockdiag(a_dst3, 1, NUM_CLASSES)

    h1, asn1, adn1, xs1 = _tc1(x, W1, A1s, A1d, Wskip1)
    xs1 = xs1 + bskip1[None, :]

    w1, den1 = _attn(asn1, adn1, src, dst)

    nums = []
    for c in range(4):
        hc = lax.slice(h1, (0, 128 * c), (N, 128 * (c + 1)))
        num = _agg(hc, src, dst, w1, 128, 2 * c, 2 * c + 1)
        nums.append(num.reshape(2, NP, 128)[:, :N, :])

    den1r = den1.reshape(2, NP, L)[:, :N, :]
    h3, asn3, adn3 = _tc2(nums, den1r, b1, xs1, Wskip2, bskip2, W3, A3s, A3d)

    w3, den3 = _attn(asn3, adn3, src, dst)
    num3 = _agg(h3, src, dst, w3, NUM_CLASSES, 0, 0)

    out = _tc3(num3.reshape(2, NP, NUM_CLASSES)[:, :N, :],
               den3.reshape(2, NP, L)[:, :N, :], b3)
    return out


# speedup vs baseline: 29.6554x; 1.3095x over previous
import jax
import jax.numpy as jnp
from jax import lax
from jax.experimental import pallas as pl
from jax.experimental.pallas import tpu as pltpu
from jax.experimental.pallas import tpu_sc as plsc

NUM_HEADS = 8
IN_CH = 128
H1 = 64
D1 = H1 * NUM_HEADS
NUM_CLASSES = 64
D3_IN = 1024
N = 10000
E = 320000

NC = 2
NS = 16
NW = NC * NS
L = 16

EPW = E // NW
BA = 80
NB = EPW // BA
NP = 10240
RPT = NP // NS

BLK = 1000


def _tc1_body(x_ref, w1_ref, as_ref, ad_ref, wsk_ref, h1_ref, asn_ref,
              adn_ref, xs1_ref):
    xb = x_ref[...]
    h = jnp.dot(xb, w1_ref[...], preferred_element_type=jnp.float32)
    h1_ref[...] = h
    asn_ref[...] = jnp.dot(h, as_ref[...], preferred_element_type=jnp.float32)
    adn_ref[...] = jnp.dot(h, ad_ref[...], preferred_element_type=jnp.float32)
    xs1_ref[...] = jnp.dot(xb, wsk_ref[...], preferred_element_type=jnp.float32)


def _tc1(x, W1, A1s, A1d, Wskip1):
    return pl.pallas_call(
        _tc1_body,
        grid=(N // BLK,),
        in_specs=[
            pl.BlockSpec((BLK, IN_CH), lambda i: (i, 0)),
            pl.BlockSpec((IN_CH, D1), lambda i: (0, 0)),
            pl.BlockSpec((D1, L), lambda i: (0, 0)),
            pl.BlockSpec((D1, L), lambda i: (0, 0)),
            pl.BlockSpec((IN_CH, D1), lambda i: (0, 0)),
        ],
        out_specs=[
            pl.BlockSpec((BLK, D1), lambda i: (i, 0)),
            pl.BlockSpec((BLK, L), lambda i: (i, 0)),
            pl.BlockSpec((BLK, L), lambda i: (i, 0)),
            pl.BlockSpec((BLK, D1), lambda i: (i, 0)),
        ],
        out_shape=[
            jax.ShapeDtypeStruct((N, D1), jnp.float32),
            jax.ShapeDtypeStruct((N, L), jnp.float32),
            jax.ShapeDtypeStruct((N, L), jnp.float32),
            jax.ShapeDtypeStruct((N, D1), jnp.float32),
        ],
    )(x, W1, A1s, A1d, Wskip1)


def _tc2_body(n0_ref, n1_ref, n2_ref, n3_ref, den_ref, b1_ref, xs1_ref,
              wsk2_ref, bsk2_ref, w3_ref, a3s_ref, a3d_ref,
              h3_ref, asn3_ref, adn3_ref):
    den = den_ref[0] + den_ref[1] + 1e-16
    parts = []
    for c, nref in enumerate((n0_ref, n1_ref, n2_ref, n3_ref)):
        num = nref[0] + nref[1]
        d2 = den[:, 2 * c:2 * c + 2]
        drep = jnp.broadcast_to(d2[:, :, None], (BLK, 2, H1)).reshape(BLK, 128)
        parts.append(num / drep)
    x1 = jnp.concatenate(parts, axis=1) + b1_ref[...][None, :]
    xs2 = jnp.dot(x1, wsk2_ref[...], preferred_element_type=jnp.float32)
    xs2 = xs2 + bsk2_ref[...][None, :]
    x3 = jnp.concatenate([xs1_ref[...], xs2], axis=1)
    h3 = jnp.dot(x3, w3_ref[...], preferred_element_type=jnp.float32)
    h3_ref[...] = h3
    asn3_ref[...] = jnp.dot(h3, a3s_ref[...], preferred_element_type=jnp.float32)
    adn3_ref[...] = jnp.dot(h3, a3d_ref[...], preferred_element_type=jnp.float32)


def _tc2(nums, den, b1, xs1, Wskip2, bskip2, W3, A3s, A3d):
    num_specs = [pl.BlockSpec((2, BLK, 128), lambda i: (0, i, 0))
                 for _ in range(4)]
    return pl.pallas_call(
        _tc2_body,
        grid=(N // BLK,),
        in_specs=num_specs + [
            pl.BlockSpec((2, BLK, L), lambda i: (0, i, 0)),
            pl.BlockSpec((D1,), lambda i: (0,)),
            pl.BlockSpec((BLK, D1), lambda i: (i, 0)),
            pl.BlockSpec((D1, D1), lambda i: (0, 0)),
            pl.BlockSpec((D1,), lambda i: (0,)),
            pl.BlockSpec((D3_IN, NUM_CLASSES), lambda i: (0, 0)),
            pl.BlockSpec((NUM_CLASSES, L), lambda i: (0, 0)),
            pl.BlockSpec((NUM_CLASSES, L), lambda i: (0, 0)),
        ],
        out_specs=[
            pl.BlockSpec((BLK, NUM_CLASSES), lambda i: (i, 0)),
            pl.BlockSpec((BLK, L), lambda i: (i, 0)),
            pl.BlockSpec((BLK, L), lambda i: (i, 0)),
        ],
        out_shape=[
            jax.ShapeDtypeStruct((N, NUM_CLASSES), jnp.float32),
            jax.ShapeDtypeStruct((N, L), jnp.float32),
            jax.ShapeDtypeStruct((N, L), jnp.float32),
        ],
    )(*nums, den, b1, xs1, Wskip2, bskip2, W3, A3s, A3d)


def _tc3_body(num_ref, den_ref, b3_ref, out_ref):
    den = den_ref[0] + den_ref[1] + 1e-16
    d = den[:, 0:1]
    drep = jnp.broadcast_to(d, (BLK, NUM_CLASSES))
    num = num_ref[0] + num_ref[1]
    out_ref[...] = num / drep + b3_ref[...][None, :]


def _tc3(num3, den3, b3):
    return pl.pallas_call(
        _tc3_body,
        grid=(N // BLK,),
        in_specs=[
            pl.BlockSpec((2, BLK, NUM_CLASSES), lambda i: (0, i, 0)),
            pl.BlockSpec((2, BLK, L), lambda i: (0, i, 0)),
            pl.BlockSpec((NUM_CLASSES,), lambda i: (0,)),
        ],
        out_specs=pl.BlockSpec((BLK, NUM_CLASSES), lambda i: (i, 0)),
        out_shape=jax.ShapeDtypeStruct((N, NUM_CLASSES), jnp.float32),
    )(num3, den3, b3)


_MESH = plsc.VectorSubcoreMesh(core_axis_name="c", subcore_axis_name="s",
                               num_cores=NC, num_subcores=NS)
_SC_PARAMS = pltpu.CompilerParams(use_tc_tiling_on_sc=False,
                                  needs_layout_passes=False)


def _zero_acc(zsrc, acc, sid, nreg):
    def zrow(i, carry):
        for k in range(nreg):
            zsrc[i, pl.ds(L * k, L)] = jnp.zeros((L,), jnp.float32)
        return carry
    lax.fori_loop(0, BA, zrow, 0)

    def zcopy(i, carry):
        pltpu.sync_copy(zsrc, acc.at[pl.ds(sid * RPT + i * BA, BA)])
        return carry
    lax.fori_loop(0, RPT // BA, zcopy, 0)


def _attn_body(asn_h, adn_h, src_h, dst_h, w_h, den_h,
               src_v, dst_v, as_v, ad_v, w_v, den_acc,
               isem, gsem, wsem, ssem):
    cid = lax.axis_index("c")
    sid = lax.axis_index("s")
    wid = sid * NC + cid
    base = wid * EPW

    def load_idx(i):
        s = lax.rem(i, 4)
        pltpu.async_copy(src_h.at[pl.ds(base + i * BA, BA)], src_v.at[s],
                         isem.at[s])
        pltpu.async_copy(dst_h.at[pl.ds(base + i * BA, BA)], dst_v.at[s],
                         isem.at[s])

    def wait_idx(i):
        s = lax.rem(i, 4)
        for _ in range(2):
            pltpu.make_async_copy(src_h.at[pl.ds(0, BA)], src_v.at[s],
                                  isem.at[s]).wait()

    def load_rows(i):
        s = lax.rem(i, 4)
        b = lax.rem(i, 2)
        pltpu.async_copy(asn_h.at[src_v.at[s]], as_v.at[b], gsem.at[b])
        pltpu.async_copy(adn_h.at[dst_v.at[s]], ad_v.at[b], gsem.at[b])

    _zero_acc(w_v.at[0], den_acc, sid, 1)
    load_idx(0)
    load_idx(1)
    wait_idx(0)
    load_rows(0)
    plsc.subcore_barrier()

    def batch(i, carry):
        s = lax.rem(i, 4)
        b = lax.rem(i, 2)

        @pl.when(i + 1 < NB)
        def _():
            wait_idx(i + 1)
            load_rows(i + 1)

        @pl.when(i >= 2)
        def _():
            pltpu.make_async_copy(w_v.at[b], w_h.at[pl.ds(0, BA)],
                                  wsem.at[b]).wait()
            pltpu.make_async_copy(w_v.at[b], den_acc.at[pl.ds(0, BA)],
                                  ssem.at[b]).wait()

        @pl.when(i + 2 < NB)
        def _():
            load_idx(i + 2)

        for _ in range(2):
            pltpu.make_async_copy(asn_h.at[pl.ds(0, BA)], as_v.at[b],
                                  gsem.at[b]).wait()

        def edge(j, c2):
            z = as_v[b, j, :] + ad_v[b, j, :]
            w_v[b, j, :] = jnp.exp(jnp.maximum(z, 0.2 * z))
            return c2
        lax.fori_loop(0, BA, edge, 0)
        pltpu.async_copy(w_v.at[b], w_h.at[pl.ds(base + i * BA, BA)],
                         wsem.at[b])
        pltpu.async_copy(w_v.at[b], den_acc.at[dst_v.at[s]], ssem.at[b],
                         add=True)
        return carry
    lax.fori_loop(0, NB, batch, 0)

    for b in range(2):
        pltpu.make_async_copy(w_v.at[b], w_h.at[pl.ds(0, BA)],
                              wsem.at[b]).wait()
        pltpu.make_async_copy(w_v.at[b], den_acc.at[pl.ds(0, BA)],
                              ssem.at[b]).wait()
    plsc.subcore_barrier()

    pltpu.sync_copy(den_acc.at[pl.ds(sid * RPT, RPT)],
                    den_h.at[pl.ds(cid * NP + sid * RPT, RPT)])


def _attn(asn, adn, src, dst):
    return pl.kernel(
        _attn_body,
        out_type=[
            jax.ShapeDtypeStruct((E, L), jnp.float32),
            jax.ShapeDtypeStruct((2 * NP, L), jnp.float32),
        ],
        mesh=_MESH,
        compiler_params=_SC_PARAMS,
        scratch_types=[
            pltpu.VMEM((4, BA), jnp.int32),
            pltpu.VMEM((4, BA), jnp.int32),
            pltpu.VMEM((2, BA, L), jnp.float32),
            pltpu.VMEM((2, BA, L), jnp.float32),
            pltpu.VMEM((2, BA, L), jnp.float32),
            pltpu.VMEM_SHARED((NP, L), jnp.float32),
            pltpu.SemaphoreType.DMA((4,)),
            pltpu.SemaphoreType.DMA((2,)),
            pltpu.SemaphoreType.DMA((2,)),
            pltpu.SemaphoreType.DMA((2,)),
        ],
    )(asn, adn, src, dst)


def _make_agg_body(F, head0, head1):
    nreg = F // L
    ngr = F // 32

    def agg_body(hc_h, src_h, dst_h, w_h, num_h,
                 src_v, dst_v, rows_v, msg_v, w_v, num_acc,
                 isem, gsem, lsem, ssem):
        cid = lax.axis_index("c")
        sid = lax.axis_index("s")
        wid = sid * NC + cid
        base = wid * EPW
        h0v = jnp.full((L,), head0, jnp.int32)
        h1v = jnp.full((L,), head1, jnp.int32)

        def load_idx(i):
            s = lax.rem(i, 4)
            pltpu.async_copy(src_h.at[pl.ds(base + i * BA, BA)], src_v.at[s],
                             isem.at[s])
            pltpu.async_copy(dst_h.at[pl.ds(base + i * BA, BA)], dst_v.at[s],
                             isem.at[s])

        def wait_idx(i):
            s = lax.rem(i, 4)
            for _ in range(2):
                pltpu.make_async_copy(src_h.at[pl.ds(0, BA)], src_v.at[s],
                                      isem.at[s]).wait()

        def load_rows(i):
            s = lax.rem(i, 4)
            b = lax.rem(i, 2)
            pltpu.async_copy(hc_h.at[src_v.at[s]], rows_v.at[b], gsem.at[b])
            pltpu.async_copy(w_h.at[pl.ds((base + i * BA), BA)], w_v.at[b],
                             lsem.at[b])

        _zero_acc(msg_v.at[0], num_acc, sid, nreg)
        load_idx(0)
        load_idx(1)
        wait_idx(0)
        load_rows(0)
        plsc.subcore_barrier()

        def batch(i, carry):
            s = lax.rem(i, 4)
            b = lax.rem(i, 2)

            @pl.when(i + 1 < NB)
            def _():
                wait_idx(i + 1)
                load_rows(i + 1)

            @pl.when(i >= 2)
            def _():
                pltpu.make_async_copy(msg_v.at[b], num_acc.at[pl.ds(0, BA)],
                                      ssem.at[b]).wait()

            @pl.when(i + 2 < NB)
            def _():
                load_idx(i + 2)

            pltpu.make_async_copy(hc_h.at[pl.ds(0, BA)], rows_v.at[b],
                                  gsem.at[b]).wait()
            pltpu.make_async_copy(w_h.at[pl.ds(0, BA)], w_v.at[b],
                                  lsem.at[b]).wait()

            def edge(j, c2):
                jj = jnp.full((L,), j, jnp.int32)
                w0 = plsc.load_gather(w_v.at[b], [jj, h0v])
                w1 = plsc.load_gather(w_v.at[b], [jj, h1v])
                for g in range(ngr):
                    wk = w0 if g < ngr // 2 else w1
                    v32 = rows_v[b, j, pl.ds(32 * g, 32)]
                    u, vv = plsc.unpack(
                        v32, format=plsc.PackFormat.INTERLEAVED,
                        preferred_element_type=jnp.float32)
                    msg_v[b, j, pl.ds(32 * g, L)] = u * wk
                    msg_v[b, j, pl.ds(32 * g + L, L)] = vv * wk
                return c2
            lax.fori_loop(0, BA, edge, 0)
            pltpu.async_copy(msg_v.at[b], num_acc.at[dst_v.at[s]],
                             ssem.at[b], add=True)
            return carry
        lax.fori_loop(0, NB, batch, 0)

        for b in range(2):
            pltpu.make_async_copy(msg_v.at[b], num_acc.at[pl.ds(0, BA)],
                                  ssem.at[b]).wait()
        plsc.subcore_barrier()

        pltpu.sync_copy(num_acc.at[pl.ds(sid * RPT, RPT)],
                        num_h.at[pl.ds(cid * NP + sid * RPT, RPT)])

    return agg_body


def _agg(hc, src, dst, w, F, head0, head1):
    return pl.kernel(
        _make_agg_body(F, head0, head1),
        out_type=jax.ShapeDtypeStruct((2 * NP, F), jnp.float32),
        mesh=_MESH,
        compiler_params=_SC_PARAMS,
        scratch_types=[
            pltpu.VMEM((4, BA), jnp.int32),
            pltpu.VMEM((4, BA), jnp.int32),
            pltpu.VMEM((2, BA, F), jnp.bfloat16),
            pltpu.VMEM((2, BA, F), jnp.float32),
            pltpu.VMEM((2, BA, L), jnp.float32),
            pltpu.VMEM_SHARED((NP, F), jnp.float32),
            pltpu.SemaphoreType.DMA((4,)),
            pltpu.SemaphoreType.DMA((2,)),
            pltpu.SemaphoreType.DMA((2,)),
            pltpu.SemaphoreType.DMA((2,)),
        ],
    )(hc, src, dst, w)


def _interleave_bf16(h):
    n, d = h.shape
    h4 = h.reshape(n, d // 32, 2, L).transpose(0, 1, 3, 2)
    return h4.reshape(n, d).astype(jnp.bfloat16)


def _blockdiag(a, heads, ch):
    eye = jnp.eye(heads, dtype=jnp.float32)
    A = (eye[:, None, :] * a[:, :, None]).reshape(heads * ch, heads)
    return jnp.concatenate(
        [A, jnp.zeros((heads * ch, L - heads), jnp.float32)], axis=1)


def kernel(x, edge_index, W1, a_src1, a_dst1, b1, Wskip1, bskip1,
           W2, a_src2, a_dst2, b2, Wskip2, bskip2,
           W3, a_src3, a_dst3, b3):
    src = edge_index[0].astype(jnp.int32)
    dst = edge_index[1].astype(jnp.int32)

    A1s = _blockdiag(a_src1, NUM_HEADS, H1)
    A1d = _blockdiag(a_dst1, NUM_HEADS, H1)
    A3s = _blockdiag(a_src3, 1, NUM_CLASSES)
    A3d = _blockdiag(a_dst3, 1, NUM_CLASSES)

    h1, asn1, adn1, xs1 = _tc1(x, W1, A1s, A1d, Wskip1)
    xs1 = xs1 + bskip1[None, :]

    w1, den1 = _attn(asn1, adn1, src, dst)

    h1b = _interleave_bf16(h1)
    nums = []
    for c in range(4):
        hc = lax.slice(h1b, (0, 128 * c), (N, 128 * (c + 1)))
        num = _agg(hc, src, dst, w1, 128, 2 * c, 2 * c + 1)
        nums.append(num.reshape(2, NP, 128)[:, :N, :])

    den1r = den1.reshape(2, NP, L)[:, :N, :]
    h3, asn3, adn3 = _tc2(nums, den1r, b1, xs1, Wskip2, bskip2, W3, A3s, A3d)

    w3, den3 = _attn(asn3, adn3, src, dst)
    num3 = _agg(_interleave_bf16(h3), src, dst, w3, NUM_CLASSES, 0, 0)

    out = _tc3(num3.reshape(2, NP, NUM_CLASSES)[:, :N, :],
               den3.reshape(2, NP, L)[:, :N, :], b3)
    return out


# speedup vs baseline: 29.8103x; 1.0052x over previous
import jax
import jax.numpy as jnp
from jax import lax
from jax.experimental import pallas as pl
from jax.experimental.pallas import tpu as pltpu
from jax.experimental.pallas import tpu_sc as plsc

NUM_HEADS = 8
IN_CH = 128
H1 = 64
D1 = H1 * NUM_HEADS
NUM_CLASSES = 64
D3_IN = 1024
N = 10000
E = 320000

NC = 2
NS = 16
NW = NC * NS
L = 16

BA = 112
EPW = 10080
NB = EPW // BA
E2 = NW * EPW
NP = 10240
RPT = NP // NS
ZB = 80

BLK = 1000


def _tc1_body(x_ref, w1_ref, as_ref, ad_ref, wsk_ref, h1_ref, asn_ref,
              adn_ref, xs1_ref):
    xb = x_ref[...]
    h = jnp.dot(xb, w1_ref[...], preferred_element_type=jnp.float32)
    h1_ref[...] = h
    asn_ref[...] = jnp.dot(h, as_ref[...], preferred_element_type=jnp.float32)
    adn_ref[...] = jnp.dot(h, ad_ref[...], preferred_element_type=jnp.float32)
    xs1_ref[...] = jnp.dot(xb, wsk_ref[...], preferred_element_type=jnp.float32)


def _tc1(x, W1, A1s, A1d, Wskip1):
    return pl.pallas_call(
        _tc1_body,
        grid=(N // BLK,),
        in_specs=[
            pl.BlockSpec((BLK, IN_CH), lambda i: (i, 0)),
            pl.BlockSpec((IN_CH, D1), lambda i: (0, 0)),
            pl.BlockSpec((D1, L), lambda i: (0, 0)),
            pl.BlockSpec((D1, L), lambda i: (0, 0)),
            pl.BlockSpec((IN_CH, D1), lambda i: (0, 0)),
        ],
        out_specs=[
            pl.BlockSpec((BLK, D1), lambda i: (i, 0)),
            pl.BlockSpec((BLK, L), lambda i: (i, 0)),
            pl.BlockSpec((BLK, L), lambda i: (i, 0)),
            pl.BlockSpec((BLK, D1), lambda i: (i, 0)),
        ],
        out_shape=[
            jax.ShapeDtypeStruct((N, D1), jnp.float32),
            jax.ShapeDtypeStruct((N, L), jnp.float32),
            jax.ShapeDtypeStruct((N, L), jnp.float32),
            jax.ShapeDtypeStruct((N, D1), jnp.float32),
        ],
    )(x, W1, A1s, A1d, Wskip1)


def _tc2_body(n0_ref, n1_ref, n2_ref, n3_ref, den_ref, b1_ref, xs1_ref,
              wsk2_ref, bsk2_ref, w3_ref, a3s_ref, a3d_ref,
              h3_ref, asn3_ref, adn3_ref):
    den = den_ref[0] + den_ref[1] + 1e-16
    parts = []
    for c, nref in enumerate((n0_ref, n1_ref, n2_ref, n3_ref)):
        num = nref[0] + nref[1]
        d2 = den[:, 2 * c:2 * c + 2]
        drep = jnp.broadcast_to(d2[:, :, None], (BLK, 2, H1)).reshape(BLK, 128)
        parts.append(num / drep)
    x1 = jnp.concatenate(parts, axis=1) + b1_ref[...][None, :]
    xs2 = jnp.dot(x1, wsk2_ref[...], preferred_element_type=jnp.float32)
    xs2 = xs2 + bsk2_ref[...][None, :]
    x3 = jnp.concatenate([xs1_ref[...], xs2], axis=1)
    h3 = jnp.dot(x3, w3_ref[...], preferred_element_type=jnp.float32)
    h3_ref[...] = h3
    asn3_ref[...] = jnp.dot(h3, a3s_ref[...], preferred_element_type=jnp.float32)
    adn3_ref[...] = jnp.dot(h3, a3d_ref[...], preferred_element_type=jnp.float32)


def _tc2(nums, den, b1, xs1, Wskip2, bskip2, W3, A3s, A3d):
    num_specs = [pl.BlockSpec((2, BLK, 128), lambda i: (0, i, 0))
                 for _ in range(4)]
    return pl.pallas_call(
        _tc2_body,
        grid=(N // BLK,),
        in_specs=num_specs + [
            pl.BlockSpec((2, BLK, L), lambda i: (0, i, 0)),
            pl.BlockSpec((D1,), lambda i: (0,)),
            pl.BlockSpec((BLK, D1), lambda i: (i, 0)),
            pl.BlockSpec((D1, D1), lambda i: (0, 0)),
            pl.BlockSpec((D1,), lambda i: (0,)),
            pl.BlockSpec((D3_IN, NUM_CLASSES), lambda i: (0, 0)),
            pl.BlockSpec((NUM_CLASSES, L), lambda i: (0, 0)),
            pl.BlockSpec((NUM_CLASSES, L), lambda i: (0, 0)),
        ],
        out_specs=[
            pl.BlockSpec((BLK, NUM_CLASSES), lambda i: (i, 0)),
            pl.BlockSpec((BLK, L), lambda i: (i, 0)),
            pl.BlockSpec((BLK, L), lambda i: (i, 0)),
        ],
        out_shape=[
            jax.ShapeDtypeStruct((N, NUM_CLASSES), jnp.float32),
            jax.ShapeDtypeStruct((N, L), jnp.float32),
            jax.ShapeDtypeStruct((N, L), jnp.float32),
        ],
    )(*nums, den, b1, xs1, Wskip2, bskip2, W3, A3s, A3d)


def _tc3_body(num_ref, den_ref, b3_ref, out_ref):
    den = den_ref[0] + den_ref[1] + 1e-16
    d = den[:, 0:1]
    drep = jnp.broadcast_to(d, (BLK, NUM_CLASSES))
    num = num_ref[0] + num_ref[1]
    out_ref[...] = num / drep + b3_ref[...][None, :]


def _tc3(num3, den3, b3):
    return pl.pallas_call(
        _tc3_body,
        grid=(N // BLK,),
        in_specs=[
            pl.BlockSpec((2, BLK, NUM_CLASSES), lambda i: (0, i, 0)),
            pl.BlockSpec((2, BLK, L), lambda i: (0, i, 0)),
            pl.BlockSpec((NUM_CLASSES,), lambda i: (0,)),
        ],
        out_specs=pl.BlockSpec((BLK, NUM_CLASSES), lambda i: (i, 0)),
        out_shape=jax.ShapeDtypeStruct((N, NUM_CLASSES), jnp.float32),
    )(num3, den3, b3)


_MESH = plsc.VectorSubcoreMesh(core_axis_name="c", subcore_axis_name="s",
                               num_cores=NC, num_subcores=NS)
_SC_PARAMS = pltpu.CompilerParams(use_tc_tiling_on_sc=False,
                                  needs_layout_passes=False)


def _zero_acc(zsrc, acc, sid, nreg):
    def zrow(i, carry):
        for k in range(nreg):
            zsrc[i, pl.ds(L * k, L)] = jnp.zeros((L,), jnp.float32)
        return carry
    lax.fori_loop(0, ZB, zrow, 0)

    def zcopy(i, carry):
        pltpu.sync_copy(zsrc.at[pl.ds(0, ZB)],
                        acc.at[pl.ds(sid * RPT + i * ZB, ZB)])
        return carry
    lax.fori_loop(0, RPT // ZB, zcopy, 0)


def _attn_body(asn_h, adn_h, src_h, dst_h, w_h, den_h,
               src_v, dst_v, as_v, ad_v, w_v, den_acc,
               isem, gsem, wsem, ssem):
    cid = lax.axis_index("c")
    sid = lax.axis_index("s")
    wid = sid * NC + cid
    base = wid * EPW

    def load_idx(i):
        s = lax.rem(i, 4)
        pltpu.async_copy(src_h.at[pl.ds(base + i * BA, BA)], src_v.at[s],
                         isem.at[s])
        pltpu.async_copy(dst_h.at[pl.ds(base + i * BA, BA)], dst_v.at[s],
                         isem.at[s])

    def wait_idx(i):
        s = lax.rem(i, 4)
        for _ in range(2):
            pltpu.make_async_copy(src_h.at[pl.ds(0, BA)], src_v.at[s],
                                  isem.at[s]).wait()

    def load_rows(i):
        s = lax.rem(i, 4)
        b = lax.rem(i, 2)
        pltpu.async_copy(asn_h.at[src_v.at[s]], as_v.at[b], gsem.at[b])
        pltpu.async_copy(adn_h.at[dst_v.at[s]], ad_v.at[b], gsem.at[b])

    _zero_acc(w_v.at[0], den_acc, sid, 1)
    load_idx(0)
    load_idx(1)
    wait_idx(0)
    load_rows(0)
    plsc.subcore_barrier()

    def batch(i, carry):
        s = lax.rem(i, 4)
        b = lax.rem(i, 2)

        @pl.when(i + 1 < NB)
        def _():
            wait_idx(i + 1)
            load_rows(i + 1)

        @pl.when(i >= 2)
        def _():
            pltpu.make_async_copy(w_v.at[b], w_h.at[pl.ds(0, BA)],
                                  wsem.at[b]).wait()
            pltpu.make_async_copy(w_v.at[b], den_acc.at[pl.ds(0, BA)],
                                  ssem.at[b]).wait()

        @pl.when(i + 2 < NB)
        def _():
            load_idx(i + 2)

        for _ in range(2):
            pltpu.make_async_copy(asn_h.at[pl.ds(0, BA)], as_v.at[b],
                                  gsem.at[b]).wait()

        def edge(j4, c2):
            for u in range(4):
                j = j4 * 4 + u
                z = as_v[b, j, :] + ad_v[b, j, :]
                w_v[b, j, :] = jnp.exp(jnp.maximum(z, 0.2 * z))
            return c2
        lax.fori_loop(0, BA // 4, edge, 0)
        pltpu.async_copy(w_v.at[b], w_h.at[pl.ds(base + i * BA, BA)],
                         wsem.at[b])
        pltpu.async_copy(w_v.at[b], den_acc.at[dst_v.at[s]], ssem.at[b],
                         add=True)
        return carry
    lax.fori_loop(0, NB, batch, 0)

    for b in range(2):
        pltpu.make_async_copy(w_v.at[b], w_h.at[pl.ds(0, BA)],
                              wsem.at[b]).wait()
        pltpu.make_async_copy(w_v.at[b], den_acc.at[pl.ds(0, BA)],
                              ssem.at[b]).wait()
    plsc.subcore_barrier()

    pltpu.sync_copy(den_acc.at[pl.ds(sid * RPT, RPT)],
                    den_h.at[pl.ds(cid * NP + sid * RPT, RPT)])


def _attn(asn, adn, src, dst):
    return pl.kernel(
        _attn_body,
        out_type=[
            jax.ShapeDtypeStruct((E2, L), jnp.float32),
            jax.ShapeDtypeStruct((2 * NP, L), jnp.float32),
        ],
        mesh=_MESH,
        compiler_params=_SC_PARAMS,
        scratch_types=[
            pltpu.VMEM((4, BA), jnp.int32),
            pltpu.VMEM((4, BA), jnp.int32),
            pltpu.VMEM((2, BA, L), jnp.float32),
            pltpu.VMEM((2, BA, L), jnp.float32),
            pltpu.VMEM((2, BA, L), jnp.float32),
            pltpu.VMEM_SHARED((NP, L), jnp.float32),
            pltpu.SemaphoreType.DMA((4,)),
            pltpu.SemaphoreType.DMA((2,)),
            pltpu.SemaphoreType.DMA((2,)),
            pltpu.SemaphoreType.DMA((2,)),
        ],
    )(asn, adn, src, dst)


def _make_agg_body(F, head0, head1):
    nreg = F // L
    ngr = F // 32

    def agg_body(hc_h, src_h, dst_h, w_h, num_h,
                 src_v, dst_v, rows_v, msg_v, w_v, num_acc,
                 isem, gsem, lsem, ssem):
        cid = lax.axis_index("c")
        sid = lax.axis_index("s")
        wid = sid * NC + cid
        base = wid * EPW
        h0v = jnp.full((L,), head0, jnp.int32)
        h1v = jnp.full((L,), head1, jnp.int32)

        def load_idx(i):
            s = lax.rem(i, 4)
            pltpu.async_copy(src_h.at[pl.ds(base + i * BA, BA)], src_v.at[s],
                             isem.at[s])
            pltpu.async_copy(dst_h.at[pl.ds(base + i * BA, BA)], dst_v.at[s],
                             isem.at[s])

        def wait_idx(i):
            s = lax.rem(i, 4)
            for _ in range(2):
                pltpu.make_async_copy(src_h.at[pl.ds(0, BA)], src_v.at[s],
                                      isem.at[s]).wait()

        def load_rows(i):
            s = lax.rem(i, 4)
            b = lax.rem(i, 2)
            pltpu.async_copy(hc_h.at[src_v.at[s]], rows_v.at[b], gsem.at[b])
            pltpu.async_copy(w_h.at[pl.ds((base + i * BA), BA)], w_v.at[b],
                             lsem.at[b])

        _zero_acc(msg_v.at[0], num_acc, sid, nreg)
        load_idx(0)
        load_idx(1)
        wait_idx(0)
        load_rows(0)
        plsc.subcore_barrier()

        def batch(i, carry):
            s = lax.rem(i, 4)
            b = lax.rem(i, 2)

            @pl.when(i + 1 < NB)
            def _():
                wait_idx(i + 1)
                load_rows(i + 1)

            @pl.when(i >= 2)
            def _():
                pltpu.make_async_copy(msg_v.at[b], num_acc.at[pl.ds(0, BA)],
                                      ssem.at[b]).wait()

            @pl.when(i + 2 < NB)
            def _():
                load_idx(i + 2)

            pltpu.make_async_copy(hc_h.at[pl.ds(0, BA)], rows_v.at[b],
                                  gsem.at[b]).wait()
            pltpu.make_async_copy(w_h.at[pl.ds(0, BA)], w_v.at[b],
                                  lsem.at[b]).wait()

            def edge(j4, c2):
                for uu in range(4):
                    j = j4 * 4 + uu
                    jj = jnp.full((L,), j, jnp.int32)
                    w0 = plsc.load_gather(w_v.at[b], [jj, h0v])
                    w1 = plsc.load_gather(w_v.at[b], [jj, h1v])
                    for g in range(ngr):
                        wk = w0 if g < ngr // 2 else w1
                        v32 = rows_v[b, j, pl.ds(32 * g, 32)]
                        u, vv = plsc.unpack(
                            v32, format=plsc.PackFormat.INTERLEAVED,
                            preferred_element_type=jnp.float32)
                        msg_v[b, j, pl.ds(32 * g, L)] = u * wk
                        msg_v[b, j, pl.ds(32 * g + L, L)] = vv * wk
                return c2
            lax.fori_loop(0, BA // 4, edge, 0)
            pltpu.async_copy(msg_v.at[b], num_acc.at[dst_v.at[s]],
                             ssem.at[b], add=True)
            return carry
        lax.fori_loop(0, NB, batch, 0)

        for b in range(2):
            pltpu.make_async_copy(msg_v.at[b], num_acc.at[pl.ds(0, BA)],
                                  ssem.at[b]).wait()
        plsc.subcore_barrier()

        pltpu.sync_copy(num_acc.at[pl.ds(sid * RPT, RPT)],
                        num_h.at[pl.ds(cid * NP + sid * RPT, RPT)])

    return agg_body


def _agg(hc, src, dst, w, F, head0, head1):
    return pl.kernel(
        _make_agg_body(F, head0, head1),
        out_type=jax.ShapeDtypeStruct((2 * NP, F), jnp.float32),
        mesh=_MESH,
        compiler_params=_SC_PARAMS,
        scratch_types=[
            pltpu.VMEM((4, BA), jnp.int32),
            pltpu.VMEM((4, BA), jnp.int32),
            pltpu.VMEM((2, BA, F), jnp.bfloat16),
            pltpu.VMEM((2, BA, F), jnp.float32),
            pltpu.VMEM((2, BA, L), jnp.float32),
            pltpu.VMEM_SHARED((NP, F), jnp.float32),
            pltpu.SemaphoreType.DMA((4,)),
            pltpu.SemaphoreType.DMA((2,)),
            pltpu.SemaphoreType.DMA((2,)),
            pltpu.SemaphoreType.DMA((2,)),
        ],
    )(hc, src, dst, w)


def _interleave_bf16(h):
    n, d = h.shape
    h4 = h.reshape(n, d // 32, 2, L).transpose(0, 1, 3, 2)
    return h4.reshape(n, d).astype(jnp.bfloat16)


def _blockdiag(a, heads, ch):
    eye = jnp.eye(heads, dtype=jnp.float32)
    A = (eye[:, None, :] * a[:, :, None]).reshape(heads * ch, heads)
    return jnp.concatenate(
        [A, jnp.zeros((heads * ch, L - heads), jnp.float32)], axis=1)


def kernel(x, edge_index, W1, a_src1, a_dst1, b1, Wskip1, bskip1,
           W2, a_src2, a_dst2, b2, Wskip2, bskip2,
           W3, a_src3, a_dst3, b3):
    pad = E2 - E
    src = jnp.concatenate(
        [edge_index[0].astype(jnp.int32), jnp.zeros((pad,), jnp.int32)])
    dst = jnp.concatenate(
        [edge_index[1].astype(jnp.int32),
         jnp.full((pad,), N, jnp.int32)])
    zpadL = jnp.zeros((NP - N, L), jnp.float32)

    A1s = _blockdiag(a_src1, NUM_HEADS, H1)
    A1d = _blockdiag(a_dst1, NUM_HEADS, H1)
    A3s = _blockdiag(a_src3, 1, NUM_CLASSES)
    A3d = _blockdiag(a_dst3, 1, NUM_CLASSES)

    h1, asn1, adn1, xs1 = _tc1(x, W1, A1s, A1d, Wskip1)
    xs1 = xs1 + bskip1[None, :]

    w1, den1 = _attn(jnp.concatenate([asn1, zpadL]),
                     jnp.concatenate([adn1, zpadL]), src, dst)

    h1b = _interleave_bf16(h1)
    nums = []
    for c in range(4):
        hc = lax.slice(h1b, (0, 128 * c), (N, 128 * (c + 1)))
        num = _agg(hc, src, dst, w1, 128, 2 * c, 2 * c + 1)
        nums.append(num.reshape(2, NP, 128)[:, :N, :])

    den1r = den1.reshape(2, NP, L)[:, :N, :]
    h3, asn3, adn3 = _tc2(nums, den1r, b1, xs1, Wskip2, bskip2, W3, A3s, A3d)

    w3, den3 = _attn(jnp.concatenate([asn3, zpadL]),
                     jnp.concatenate([adn3, zpadL]), src, dst)
    num3 = _agg(_interleave_bf16(h3), src, dst, w3, NUM_CLASSES, 0, 0)

    out = _tc3(num3.reshape(2, NP, NUM_CLASSES)[:, :N, :],
               den3.reshape(2, NP, L)[:, :N, :], b3)
    return out


# speedup vs baseline: 30.0340x; 1.0075x over previous
import jax
import jax.numpy as jnp
from jax import lax
from jax.experimental import pallas as pl
from jax.experimental.pallas import tpu as pltpu
from jax.experimental.pallas import tpu_sc as plsc

NUM_HEADS = 8
IN_CH = 128
H1 = 64
D1 = H1 * NUM_HEADS
NUM_CLASSES = 64
D3_IN = 1024
N = 10000
E = 320000

NC = 2
NS = 16
NW = NC * NS
L = 16

BA = 112
EPW = 10080
NB = EPW // BA
E2 = NW * EPW
NP = 10240
RPT = NP // NS
ZB = 80

BLK = 1000


def _tc1_body(x_ref, w1_ref, as_ref, ad_ref, wsk_ref, h1_ref, asn_ref,
              adn_ref, xs1_ref):
    xb = x_ref[...]
    h = jnp.dot(xb, w1_ref[...], preferred_element_type=jnp.float32)
    h1_ref[...] = h
    asn_ref[...] = jnp.dot(h, as_ref[...], preferred_element_type=jnp.float32)
    adn_ref[...] = jnp.dot(h, ad_ref[...], preferred_element_type=jnp.float32)
    xs1_ref[...] = jnp.dot(xb, wsk_ref[...], preferred_element_type=jnp.float32)


def _tc1(x, W1, A1s, A1d, Wskip1):
    return pl.pallas_call(
        _tc1_body,
        grid=(N // BLK,),
        in_specs=[
            pl.BlockSpec((BLK, IN_CH), lambda i: (i, 0)),
            pl.BlockSpec((IN_CH, D1), lambda i: (0, 0)),
            pl.BlockSpec((D1, L), lambda i: (0, 0)),
            pl.BlockSpec((D1, L), lambda i: (0, 0)),
            pl.BlockSpec((IN_CH, D1), lambda i: (0, 0)),
        ],
        out_specs=[
            pl.BlockSpec((BLK, D1), lambda i: (i, 0)),
            pl.BlockSpec((BLK, L), lambda i: (i, 0)),
            pl.BlockSpec((BLK, L), lambda i: (i, 0)),
            pl.BlockSpec((BLK, D1), lambda i: (i, 0)),
        ],
        out_shape=[
            jax.ShapeDtypeStruct((N, D1), jnp.float32),
            jax.ShapeDtypeStruct((N, L), jnp.float32),
            jax.ShapeDtypeStruct((N, L), jnp.float32),
            jax.ShapeDtypeStruct((N, D1), jnp.float32),
        ],
    )(x, W1, A1s, A1d, Wskip1)


def _tc2_body(n0_ref, n1_ref, n2_ref, n3_ref, den_ref, b1_ref, xs1_ref,
              wsk2_ref, bsk2_ref, w3_ref, a3s_ref, a3d_ref,
              h3_ref, asn3_ref, adn3_ref):
    den = den_ref[0] + den_ref[1] + 1e-16
    parts = []
    for c, nref in enumerate((n0_ref, n1_ref, n2_ref, n3_ref)):
        num = nref[0] + nref[1]
        d2 = den[:, 2 * c:2 * c + 2]
        drep = jnp.broadcast_to(d2[:, :, None], (BLK, 2, H1)).reshape(BLK, 128)
        parts.append(num / drep)
    x1 = jnp.concatenate(parts, axis=1) + b1_ref[...][None, :]
    xs2 = jnp.dot(x1, wsk2_ref[...], preferred_element_type=jnp.float32)
    xs2 = xs2 + bsk2_ref[...][None, :]
    x3 = jnp.concatenate([xs1_ref[...], xs2], axis=1)
    h3 = jnp.dot(x3, w3_ref[...], preferred_element_type=jnp.float32)
    h3_ref[...] = h3
    asn3_ref[...] = jnp.dot(h3, a3s_ref[...], preferred_element_type=jnp.float32)
    adn3_ref[...] = jnp.dot(h3, a3d_ref[...], preferred_element_type=jnp.float32)


def _tc2(nums, den, b1, xs1, Wskip2, bskip2, W3, A3s, A3d):
    num_specs = [pl.BlockSpec((2, BLK, 128), lambda i: (0, i, 0))
                 for _ in range(4)]
    return pl.pallas_call(
        _tc2_body,
        grid=(N // BLK,),
        in_specs=num_specs + [
            pl.BlockSpec((2, BLK, L), lambda i: (0, i, 0)),
            pl.BlockSpec((D1,), lambda i: (0,)),
            pl.BlockSpec((BLK, D1), lambda i: (i, 0)),
            pl.BlockSpec((D1, D1), lambda i: (0, 0)),
            pl.BlockSpec((D1,), lambda i: (0,)),
            pl.BlockSpec((D3_IN, NUM_CLASSES), lambda i: (0, 0)),
            pl.BlockSpec((NUM_CLASSES, L), lambda i: (0, 0)),
            pl.BlockSpec((NUM_CLASSES, L), lambda i: (0, 0)),
        ],
        out_specs=[
            pl.BlockSpec((BLK, NUM_CLASSES), lambda i: (i, 0)),
            pl.BlockSpec((BLK, L), lambda i: (i, 0)),
            pl.BlockSpec((BLK, L), lambda i: (i, 0)),
        ],
        out_shape=[
            jax.ShapeDtypeStruct((N, NUM_CLASSES), jnp.float32),
            jax.ShapeDtypeStruct((N, L), jnp.float32),
            jax.ShapeDtypeStruct((N, L), jnp.float32),
        ],
    )(*nums, den, b1, xs1, Wskip2, bskip2, W3, A3s, A3d)


def _tc3_body(num_ref, den_ref, b3_ref, out_ref):
    den = den_ref[0] + den_ref[1] + 1e-16
    d = den[:, 0:1]
    drep = jnp.broadcast_to(d, (BLK, NUM_CLASSES))
    num = num_ref[0] + num_ref[1]
    out_ref[...] = num / drep + b3_ref[...][None, :]


def _tc3(num3, den3, b3):
    return pl.pallas_call(
        _tc3_body,
        grid=(N // BLK,),
        in_specs=[
            pl.BlockSpec((2, BLK, NUM_CLASSES), lambda i: (0, i, 0)),
            pl.BlockSpec((2, BLK, L), lambda i: (0, i, 0)),
            pl.BlockSpec((NUM_CLASSES,), lambda i: (0,)),
        ],
        out_specs=pl.BlockSpec((BLK, NUM_CLASSES), lambda i: (i, 0)),
        out_shape=jax.ShapeDtypeStruct((N, NUM_CLASSES), jnp.float32),
    )(num3, den3, b3)


_MESH = plsc.VectorSubcoreMesh(core_axis_name="c", subcore_axis_name="s",
                               num_cores=NC, num_subcores=NS)
_SC_PARAMS = pltpu.CompilerParams(use_tc_tiling_on_sc=False,
                                  needs_layout_passes=False)


def _zero_acc(zsrc, acc, sid, nreg):
    def zrow(i, carry):
        for k in range(nreg):
            zsrc[i, pl.ds(L * k, L)] = jnp.zeros((L,), jnp.float32)
        return carry
    lax.fori_loop(0, ZB, zrow, 0)

    def zcopy(i, carry):
        pltpu.sync_copy(zsrc.at[pl.ds(0, ZB)],
                        acc.at[pl.ds(sid * RPT + i * ZB, ZB)])
        return carry
    lax.fori_loop(0, RPT // ZB, zcopy, 0)


def _attn_body(asn_h, adn_h, src_h, dst_h, w_h, den_h,
               src_v, dst_v, as_v, ad_v, w_v, den_acc,
               isem, gsem, wsem, ssem):
    cid = lax.axis_index("c")
    sid = lax.axis_index("s")
    wid = sid * NC + cid
    base = wid * EPW

    def load_idx(i):
        s = lax.rem(i, 4)
        pltpu.async_copy(src_h.at[pl.ds(base + i * BA, BA)], src_v.at[s],
                         isem.at[s])
        pltpu.async_copy(dst_h.at[pl.ds(base + i * BA, BA)], dst_v.at[s],
                         isem.at[s])

    def wait_idx(i):
        s = lax.rem(i, 4)
        for _ in range(2):
            pltpu.make_async_copy(src_h.at[pl.ds(0, BA)], src_v.at[s],
                                  isem.at[s]).wait()

    def load_rows(i):
        s = lax.rem(i, 4)
        b = lax.rem(i, 2)
        pltpu.async_copy(asn_h.at[src_v.at[s]], as_v.at[b], gsem.at[b])
        pltpu.async_copy(adn_h.at[dst_v.at[s]], ad_v.at[b], gsem.at[b])

    _zero_acc(w_v.at[0], den_acc, sid, 1)
    load_idx(0)
    load_idx(1)
    wait_idx(0)
    load_rows(0)
    plsc.subcore_barrier()

    def batch(i, carry):
        s = lax.rem(i, 4)
        b = lax.rem(i, 2)

        @pl.when(i + 1 < NB)
        def _():
            wait_idx(i + 1)
            load_rows(i + 1)

        @pl.when(i >= 2)
        def _():
            pltpu.make_async_copy(w_v.at[b], w_h.at[pl.ds(0, BA)],
                                  wsem.at[b]).wait()
            pltpu.make_async_copy(w_v.at[b], den_acc.at[pl.ds(0, BA)],
                                  ssem.at[b]).wait()

        @pl.when(i + 2 < NB)
        def _():
            load_idx(i + 2)

        for _ in range(2):
            pltpu.make_async_copy(asn_h.at[pl.ds(0, BA)], as_v.at[b],
                                  gsem.at[b]).wait()

        def edge(j4, c2):
            for u in range(4):
                j = j4 * 4 + u
                z = as_v[b, j, :] + ad_v[b, j, :]
                w_v[b, j, :] = jnp.exp(jnp.maximum(z, 0.2 * z))
            return c2
        lax.fori_loop(0, BA // 4, edge, 0)
        pltpu.async_copy(w_v.at[b], w_h.at[pl.ds(base + i * BA, BA)],
                         wsem.at[b])
        pltpu.async_copy(w_v.at[b], den_acc.at[dst_v.at[s]], ssem.at[b],
                         add=True)
        return carry
    lax.fori_loop(0, NB, batch, 0)

    for b in range(2):
        pltpu.make_async_copy(w_v.at[b], w_h.at[pl.ds(0, BA)],
                              wsem.at[b]).wait()
        pltpu.make_async_copy(w_v.at[b], den_acc.at[pl.ds(0, BA)],
                              ssem.at[b]).wait()
    plsc.subcore_barrier()

    pltpu.sync_copy(den_acc.at[pl.ds(sid * RPT, RPT)],
                    den_h.at[pl.ds(cid * NP + sid * RPT, RPT)])


def _attn(asn, adn, src, dst):
    return pl.kernel(
        _attn_body,
        out_type=[
            jax.ShapeDtypeStruct((E2, L), jnp.float32),
            jax.ShapeDtypeStruct((2 * NP, L), jnp.float32),
        ],
        mesh=_MESH,
        compiler_params=_SC_PARAMS,
        scratch_types=[
            pltpu.VMEM((4, BA), jnp.int32),
            pltpu.VMEM((4, BA), jnp.int32),
            pltpu.VMEM((2, BA, L), jnp.float32),
            pltpu.VMEM((2, BA, L), jnp.float32),
            pltpu.VMEM((2, BA, L), jnp.float32),
            pltpu.VMEM_SHARED((NP, L), jnp.float32),
            pltpu.SemaphoreType.DMA((4,)),
            pltpu.SemaphoreType.DMA((2,)),
            pltpu.SemaphoreType.DMA((2,)),
            pltpu.SemaphoreType.DMA((2,)),
        ],
    )(asn, adn, src, dst)


def _make_agg_body(F, head0, head1):
    nreg = F // L
    ngr = F // 32

    def agg_body(hc_h, src_h, dst_h, w_h, num_h,
                 src_v, dst_v, rows_v, msg_v, w_v, num_acc,
                 isem, gsem, lsem, ssem):
        cid = lax.axis_index("c")
        sid = lax.axis_index("s")
        wid = sid * NC + cid
        base = wid * EPW
        h0v = jnp.full((L,), head0, jnp.int32)
        h1v = jnp.full((L,), head1, jnp.int32)

        def load_idx(i):
            s = lax.rem(i, 4)
            pltpu.async_copy(src_h.at[pl.ds(base + i * BA, BA)], src_v.at[s],
                             isem.at[s])
            pltpu.async_copy(dst_h.at[pl.ds(base + i * BA, BA)], dst_v.at[s],
                             isem.at[s])

        def wait_idx(i):
            s = lax.rem(i, 4)
            for _ in range(2):
                pltpu.make_async_copy(src_h.at[pl.ds(0, BA)], src_v.at[s],
                                      isem.at[s]).wait()

        def load_rows(i):
            s = lax.rem(i, 4)
            b = lax.rem(i, 2)
            pltpu.async_copy(hc_h.at[src_v.at[s]], rows_v.at[b], gsem.at[b])
            pltpu.async_copy(w_h.at[pl.ds((base + i * BA), BA)], w_v.at[b],
                             lsem.at[b])

        _zero_acc(msg_v.at[0], num_acc, sid, nreg)
        load_idx(0)
        load_idx(1)
        wait_idx(0)
        load_rows(0)
        plsc.subcore_barrier()

        def batch(i, carry):
            s = lax.rem(i, 4)
            b = lax.rem(i, 2)

            @pl.when(i + 1 < NB)
            def _():
                wait_idx(i + 1)
                load_rows(i + 1)

            @pl.when(i >= 2)
            def _():
                pltpu.make_async_copy(msg_v.at[b], num_acc.at[pl.ds(0, BA)],
                                      ssem.at[b]).wait()

            @pl.when(i + 2 < NB)
            def _():
                load_idx(i + 2)

            pltpu.make_async_copy(hc_h.at[pl.ds(0, BA)], rows_v.at[b],
                                  gsem.at[b]).wait()
            pltpu.make_async_copy(w_h.at[pl.ds(0, BA)], w_v.at[b],
                                  lsem.at[b]).wait()

            def edge(j4, c2):
                for uu in range(4):
                    j = j4 * 4 + uu
                    jj = jnp.full((L,), j, jnp.int32)
                    w0 = plsc.load_gather(w_v.at[b], [jj, h0v])
                    w1 = plsc.load_gather(w_v.at[b], [jj, h1v])
                    for g in range(ngr):
                        wk = w0 if g < ngr // 2 else w1
                        v32 = rows_v[b, j, pl.ds(32 * g, 32)]
                        u, vv = plsc.unpack(
                            v32, format=plsc.PackFormat.INTERLEAVED,
                            preferred_element_type=jnp.float32)
                        msg_v[b, j, pl.ds(32 * g, L)] = u * wk
                        msg_v[b, j, pl.ds(32 * g + L, L)] = vv * wk
                return c2
            lax.fori_loop(0, BA // 4, edge, 0)
            pltpu.async_copy(msg_v.at[b], num_acc.at[dst_v.at[s]],
                             ssem.at[b], add=True)
            return carry
        lax.fori_loop(0, NB, batch, 0)

        for b in range(2):
            pltpu.make_async_copy(msg_v.at[b], num_acc.at[pl.ds(0, BA)],
                                  ssem.at[b]).wait()
        plsc.subcore_barrier()

        pltpu.sync_copy(num_acc.at[pl.ds(sid * RPT, RPT)],
                        num_h.at[pl.ds(cid * NP + sid * RPT, RPT)])

    return agg_body


def _agg(hc, src, dst, w, F, head0, head1):
    return pl.kernel(
        _make_agg_body(F, head0, head1),
        out_type=jax.ShapeDtypeStruct((2 * NP, F), jnp.float32),
        mesh=_MESH,
        compiler_params=_SC_PARAMS,
        scratch_types=[
            pltpu.VMEM((4, BA), jnp.int32),
            pltpu.VMEM((4, BA), jnp.int32),
            pltpu.VMEM((2, BA, F), jnp.bfloat16),
            pltpu.VMEM((2, BA, F), jnp.float32),
            pltpu.VMEM((2, BA, L), jnp.float32),
            pltpu.VMEM_SHARED((NP, F), jnp.float32),
            pltpu.SemaphoreType.DMA((4,)),
            pltpu.SemaphoreType.DMA((2,)),
            pltpu.SemaphoreType.DMA((2,)),
            pltpu.SemaphoreType.DMA((2,)),
        ],
    )(hc, src, dst, w)


def _attn_agg3_body(asn_h, adn_h, hc_h, src_h, dst_h, num_h, den_h,
                    src_v, dst_v, as_v, ad_v, w_v, rows_v, msg_v,
                    num_acc, den_acc, isem, gsem, ssem, dsem):
    F = NUM_CLASSES
    ngr = F // 32
    cid = lax.axis_index("c")
    sid = lax.axis_index("s")
    wid = sid * NC + cid
    base = wid * EPW

    def load_idx(i):
        s = lax.rem(i, 4)
        pltpu.async_copy(src_h.at[pl.ds(base + i * BA, BA)], src_v.at[s],
                         isem.at[s])
        pltpu.async_copy(dst_h.at[pl.ds(base + i * BA, BA)], dst_v.at[s],
                         isem.at[s])

    def wait_idx(i):
        s = lax.rem(i, 4)
        for _ in range(2):
            pltpu.make_async_copy(src_h.at[pl.ds(0, BA)], src_v.at[s],
                                  isem.at[s]).wait()

    def load_rows(i):
        s = lax.rem(i, 4)
        b = lax.rem(i, 2)
        pltpu.async_copy(asn_h.at[src_v.at[s]], as_v.at[b], gsem.at[b])
        pltpu.async_copy(adn_h.at[dst_v.at[s]], ad_v.at[b], gsem.at[b])
        pltpu.async_copy(hc_h.at[src_v.at[s]], rows_v.at[b], gsem.at[b])

    _zero_acc(msg_v.at[0], num_acc, sid, F // L)
    _zero_acc(w_v.at[0], den_acc, sid, 1)
    load_idx(0)
    load_idx(1)
    wait_idx(0)
    load_rows(0)
    plsc.subcore_barrier()

    def batch(i, carry):
        s = lax.rem(i, 4)
        b = lax.rem(i, 2)

        @pl.when(i + 1 < NB)
        def _():
            wait_idx(i + 1)
            load_rows(i + 1)

        @pl.when(i >= 2)
        def _():
            pltpu.make_async_copy(msg_v.at[b], num_acc.at[pl.ds(0, BA)],
                                  ssem.at[b]).wait()
            pltpu.make_async_copy(w_v.at[b], den_acc.at[pl.ds(0, BA)],
                                  dsem.at[b]).wait()

        @pl.when(i + 2 < NB)
        def _():
            load_idx(i + 2)

        for _ in range(3):
            pltpu.make_async_copy(asn_h.at[pl.ds(0, BA)], as_v.at[b],
                                  gsem.at[b]).wait()

        def edge(j4, c2):
            for uu in range(4):
                j = j4 * 4 + uu
                z = as_v[b, j, :] + ad_v[b, j, :]
                w = jnp.exp(jnp.maximum(z, 0.2 * z))
                w_v[b, j, :] = w
                for g in range(ngr):
                    v32 = rows_v[b, j, pl.ds(32 * g, 32)]
                    u, vv = plsc.unpack(
                        v32, format=plsc.PackFormat.INTERLEAVED,
                        preferred_element_type=jnp.float32)
                    msg_v[b, j, pl.ds(32 * g, L)] = u * w
                    msg_v[b, j, pl.ds(32 * g + L, L)] = vv * w
            return c2
        lax.fori_loop(0, BA // 4, edge, 0)
        pltpu.async_copy(msg_v.at[b], num_acc.at[dst_v.at[s]],
                         ssem.at[b], add=True)
        pltpu.async_copy(w_v.at[b], den_acc.at[dst_v.at[s]],
                         dsem.at[b], add=True)
        return carry
    lax.fori_loop(0, NB, batch, 0)

    for b in range(2):
        pltpu.make_async_copy(msg_v.at[b], num_acc.at[pl.ds(0, BA)],
                              ssem.at[b]).wait()
        pltpu.make_async_copy(w_v.at[b], den_acc.at[pl.ds(0, BA)],
                              dsem.at[b]).wait()
    plsc.subcore_barrier()

    pltpu.sync_copy(num_acc.at[pl.ds(sid * RPT, RPT)],
                    num_h.at[pl.ds(cid * NP + sid * RPT, RPT)])
    pltpu.sync_copy(den_acc.at[pl.ds(sid * RPT, RPT)],
                    den_h.at[pl.ds(cid * NP + sid * RPT, RPT)])


def _attn_agg3(asn, adn, hc, src, dst):
    F = NUM_CLASSES
    return pl.kernel(
        _attn_agg3_body,
        out_type=[
            jax.ShapeDtypeStruct((2 * NP, F), jnp.float32),
            jax.ShapeDtypeStruct((2 * NP, L), jnp.float32),
        ],
        mesh=_MESH,
        compiler_params=_SC_PARAMS,
        scratch_types=[
            pltpu.VMEM((4, BA), jnp.int32),
            pltpu.VMEM((4, BA), jnp.int32),
            pltpu.VMEM((2, BA, L), jnp.float32),
            pltpu.VMEM((2, BA, L), jnp.float32),
            pltpu.VMEM((2, BA, L), jnp.float32),
            pltpu.VMEM((2, BA, F), jnp.bfloat16),
            pltpu.VMEM((2, BA, F), jnp.float32),
            pltpu.VMEM_SHARED((NP, F), jnp.float32),
            pltpu.VMEM_SHARED((NP, L), jnp.float32),
            pltpu.SemaphoreType.DMA((4,)),
            pltpu.SemaphoreType.DMA((2,)),
            pltpu.SemaphoreType.DMA((2,)),
            pltpu.SemaphoreType.DMA((2,)),
        ],
    )(asn, adn, hc, src, dst)


def _interleave_bf16(h):
    n, d = h.shape
    h4 = h.reshape(n, d // 32, 2, L).transpose(0, 1, 3, 2)
    return h4.reshape(n, d).astype(jnp.bfloat16)


def _blockdiag(a, heads, ch):
    eye = jnp.eye(heads, dtype=jnp.float32)
    A = (eye[:, None, :] * a[:, :, None]).reshape(heads * ch, heads)
    return jnp.concatenate(
        [A, jnp.zeros((heads * ch, L - heads), jnp.float32)], axis=1)


def kernel(x, edge_index, W1, a_src1, a_dst1, b1, Wskip1, bskip1,
           W2, a_src2, a_dst2, b2, Wskip2, bskip2,
           W3, a_src3, a_dst3, b3):
    pad = E2 - E
    src = jnp.concatenate(
        [edge_index[0].astype(jnp.int32), jnp.zeros((pad,), jnp.int32)])
    dst = jnp.concatenate(
        [edge_index[1].astype(jnp.int32),
         jnp.full((pad,), N, jnp.int32)])
    zpadL = jnp.zeros((NP - N, L), jnp.float32)

    A1s = _blockdiag(a_src1, NUM_HEADS, H1)
    A1d = _blockdiag(a_dst1, NUM_HEADS, H1)
    A3s = _blockdiag(a_src3, 1, NUM_CLASSES)
    A3d = _blockdiag(a_dst3, 1, NUM_CLASSES)

    h1, asn1, adn1, xs1 = _tc1(x, W1, A1s, A1d, Wskip1)
    xs1 = xs1 + bskip1[None, :]

    w1, den1 = _attn(jnp.concatenate([asn1, zpadL]),
                     jnp.concatenate([adn1, zpadL]), src, dst)

    h1b = _interleave_bf16(h1)
    nums = []
    for c in range(4):
        hc = lax.slice(h1b, (0, 128 * c), (N, 128 * (c + 1)))
        num = _agg(hc, src, dst, w1, 128, 2 * c, 2 * c + 1)
        nums.append(num.reshape(2, NP, 128)[:, :N, :])

    den1r = den1.reshape(2, NP, L)[:, :N, :]
    h3, asn3, adn3 = _tc2(nums, den1r, b1, xs1, Wskip2, bskip2, W3, A3s, A3d)

    num3, den3 = _attn_agg3(jnp.concatenate([asn3, zpadL]),
                            jnp.concatenate([adn3, zpadL]),
                            _interleave_bf16(h3), src, dst)

    out = _tc3(num3.reshape(2, NP, NUM_CLASSES)[:, :N, :],
               den3.reshape(2, NP, L)[:, :N, :], b3)
    return out
